# baseline (speedup 1.0000x reference)
import functools

import jax
import jax.numpy as jnp
from jax import lax
from jax.experimental import pallas as pl
from jax.experimental.pallas import tpu as pltpu
from jax.experimental.pallas import tpu_sc as plsc

BATCH = 4096
CON = 13
CAT = 26
VOCAB = 100000
DIM = 64
ROWS = 1 + CON + CAT

NUM_CORES = 2
NUM_SUBCORES = 16
NW = NUM_CORES * NUM_SUBCORES
BPW = BATCH // NW
SUB = 32
NSUB = BPW // SUB


def _body(xcon_hbm, idx_hbm, cls_hbm, conW_hbm, conb_hbm, tables_hbm,
          out_hbm, idx_v, xcon_v, cls_v, conW_v, conb_v, catbuf, densebuf,
          gsem):
    w = lax.axis_index("s") * NUM_CORES + lax.axis_index("c")
    b0 = pl.multiple_of(w * BPW, BPW)

    pltpu.sync_copy(idx_hbm.at[pl.ds(b0, BPW), :], idx_v)
    pltpu.sync_copy(xcon_hbm.at[pl.ds(b0 * CON, BPW * CON)], xcon_v)
    pltpu.sync_copy(cls_hbm, cls_v)
    pltpu.sync_copy(conW_hbm, conW_v)
    pltpu.sync_copy(conb_hbm, conb_v)

    cls_regs = [cls_v[pl.ds(q * 16, 16)] for q in range(DIM // 16)]

    for c in range(NSUB):
        bb = c * SUB
        handles = [
            pltpu.async_copy(tables_hbm.at[idx_v.at[bb + j]], catbuf.at[j],
                             gsem)
            for j in range(SUB)
        ]

        def cls_body(b, _):
            for q in range(DIM // 16):
                densebuf[b, 0, pl.ds(q * 16, 16)] = cls_regs[q]
            return _

        lax.fori_loop(0, SUB, cls_body, None)

        for f in range(CON):
            w_regs = [conW_v[f, pl.ds(q * 16, 16)] for q in range(DIM // 16)]
            b_regs = [conb_v[f, pl.ds(q * 16, 16)] for q in range(DIM // 16)]

            def con_body(b, _, f=f, w_regs=w_regs, b_regs=b_regs):
                xb = plsc.load_gather(
                    xcon_v,
                    [jnp.full((16,), (bb + b) * CON + f, dtype=jnp.int32)])
                for q in range(DIM // 16):
                    densebuf[b, 1 + f, pl.ds(q * 16, 16)] = jnp.maximum(
                        xb * w_regs[q] + b_regs[q], 0.0)
                return _

            lax.fori_loop(0, SUB, con_body, None)

        for h in handles:
            h.wait()

        pltpu.sync_copy(densebuf,
                        out_hbm.at[pl.ds(b0 + bb, SUB), pl.ds(0, 1 + CON), :])
        pltpu.sync_copy(catbuf,
                        out_hbm.at[pl.ds(b0 + bb, SUB), pl.ds(1 + CON, CAT), :])


@jax.jit
def _sc_call(xcon, flat_idx, cls_flat, conW, conb, tables_flat):
    mesh = plsc.VectorSubcoreMesh(core_axis_name="c", subcore_axis_name="s")
    kern = pl.kernel(
        _body,
        out_type=jax.ShapeDtypeStruct((BATCH, ROWS, DIM), jnp.float32),
        mesh=mesh,
        compiler_params=pltpu.CompilerParams(use_tc_tiling_on_sc=False,
                                             needs_layout_passes=False),
        scratch_types=[
            pltpu.VMEM((BPW, CAT), jnp.int32),
            pltpu.VMEM((BPW * CON,), jnp.float32),
            pltpu.VMEM((DIM,), jnp.float32),
            pltpu.VMEM((CON, DIM), jnp.float32),
            pltpu.VMEM((CON, DIM), jnp.float32),
            pltpu.VMEM((SUB, CAT, DIM), jnp.float32),
            pltpu.VMEM((SUB, 1 + CON, DIM), jnp.float32),
            pltpu.SemaphoreType.DMA,
        ],
    )
    return kern(xcon.reshape(BATCH * CON), flat_idx, cls_flat, conW, conb,
                tables_flat)


def kernel(x_con, x_cat, cls, con_W, con_b, tables):
    offs = (jnp.arange(CAT, dtype=jnp.int32) * VOCAB)[None, :]
    flat_idx = x_cat.astype(jnp.int32) + offs
    tables_flat = tables.reshape(CAT * VOCAB, DIM)
    return _sc_call(x_con, flat_idx, cls.reshape(DIM), con_W, con_b,
                    tables_flat)
